# baseline (speedup 1.0000x reference)
import jax
import jax.numpy as jnp
from jax import lax
from jax.experimental import pallas as pl
from jax.experimental.pallas import tpu as pltpu

_E = 8
_H = 768
_BT = 1024
_NBUF = 8


def _top2(logits):
    lt = logits.T
    sub = lax.broadcasted_iota(jnp.int32, lt.shape, 0)
    m1 = jnp.max(lt, axis=0, keepdims=True)
    i1 = jnp.min(jnp.where(lt == m1, sub, _E), axis=0, keepdims=True)
    masked = jnp.where(sub == i1, -jnp.inf, lt)
    m2 = jnp.max(masked, axis=0, keepdims=True)
    i2 = jnp.min(jnp.where(masked == m2, sub, _E), axis=0, keepdims=True)
    e = jnp.exp(m2 - m1)
    den = 1.0 + e
    return (jnp.concatenate([1.0 / den, e / den], axis=0),
            jnp.concatenate([i1, i2], axis=0))


def _router(n_tokens, x_hbm, w_ref, logits_ref, prob_ref, idx_ref, buf, sems):
    nblk = n_tokens // _BT

    def start(b):
        slot = b % _NBUF
        pltpu.make_async_copy(
            x_hbm.at[pl.ds(b * _BT, _BT), :], buf.at[slot], sems.at[slot]
        ).start()

    for b in range(_NBUF - 1):
        start(b)
    w = w_ref[...]
    for b in range(nblk):
        slot = b % _NBUF
        pltpu.make_async_copy(
            x_hbm.at[pl.ds(b * _BT, _BT), :], buf.at[slot], sems.at[slot]
        ).wait()
        if b + _NBUF - 1 < nblk:
            start(b + _NBUF - 1)
        logits = lax.dot_general(
            buf[slot], w, (((1,), (1,)), ((), ())),
            preferred_element_type=jnp.float32,
        )
        logits_ref[pl.ds(b * _BT, _BT), :] = logits
        prob, idx = _top2(logits)
        prob_ref[:, pl.ds(b * _BT, _BT)] = prob
        idx_ref[:, pl.ds(b * _BT, _BT)] = idx


def kernel(hidden_states, weight):
    n_tokens, hidden = hidden_states.shape
    import functools
    logits, prob_t, idx_t = pl.pallas_call(
        functools.partial(_router, n_tokens),
        in_specs=[
            pl.BlockSpec(memory_space=pl.ANY),
            pl.BlockSpec(memory_space=pltpu.VMEM),
        ],
        out_specs=[
            pl.BlockSpec(memory_space=pltpu.VMEM),
            pl.BlockSpec(memory_space=pltpu.VMEM),
            pl.BlockSpec(memory_space=pltpu.VMEM),
        ],
        out_shape=[
            jax.ShapeDtypeStruct((n_tokens, _E), jnp.float32),
            jax.ShapeDtypeStruct((2, n_tokens), jnp.float32),
            jax.ShapeDtypeStruct((2, n_tokens), jnp.int32),
        ],
        scratch_shapes=[
            pltpu.VMEM((_NBUF, _BT, _H), jnp.float32),
            pltpu.SemaphoreType.DMA((_NBUF,)),
        ],
    )(hidden_states, weight)
    return (logits, prob_t.T, idx_t.T)
